# Initial kernel scaffold; baseline (speedup 1.0000x reference)
import jax
import jax.numpy as jnp
from jax.experimental import pallas as pl


def kernel(mu, t, node_context, global_cond, edge_index, params):
    raise NotImplementedError("write your pallas kernel here")


# speedup vs baseline: 2.5104x; 2.5104x over previous
import functools

import jax
import jax.numpy as jnp
from jax import lax
from jax.experimental import pallas as pl
from jax.experimental.pallas import tpu as pltpu
from jax.experimental.pallas import tpu_sc as plsc

B, N, E, H = 8, 1024, 16384, 128
BN = B * N
BE = B * E
BH = B * H
TOT = B * N * N
PAD = 128
f32 = jnp.float32
i32 = jnp.int32


def _silu(x):
    return x * (1.0 / (1.0 + jnp.exp(-x)))


def _glob_embed_body(gc, w1, b1, w2, b2, o):
    x = jnp.maximum(gc[...] @ w1[...] + b1[...], 0.0)
    o[...] = x @ w2[...] + b2[...]


def _glob_embed(gc, W1, b1, W2, b2):
    return pl.pallas_call(
        _glob_embed_body,
        out_shape=jax.ShapeDtypeStruct((B, H), f32),
    )(gc, W1, b1.reshape(1, -1), W2, b2.reshape(1, -1))


def _proj_body(h, wt, wb, bb, ps, pd):
    hv = h[...]
    ps[...] = hv @ wt[...]
    pd[...] = hv @ wb[...] + bb[...]


def _proj(h, Wt, Wb, bb):
    ind = h.shape[1]
    blk = 1024
    return pl.pallas_call(
        _proj_body,
        grid=(BN // blk,),
        in_specs=[
            pl.BlockSpec((blk, ind), lambda i: (i, 0)),
            pl.BlockSpec((ind, H), lambda i: (0, 0)),
            pl.BlockSpec((ind, H), lambda i: (0, 0)),
            pl.BlockSpec((1, H), lambda i: (0, 0)),
        ],
        out_specs=[
            pl.BlockSpec((blk, H), lambda i: (i, 0)),
            pl.BlockSpec((blk, H), lambda i: (i, 0)),
        ],
        out_shape=[jax.ShapeDtypeStruct((BN, H), f32)] * 2,
    )(h, Wt, Wb, bb.reshape(1, -1))


def _edgemlp_body(ga, gb, w, b, hi, lo):
    x = _silu(ga[...] + gb[...])
    m = x @ w[...] + b[...]
    mh = m.astype(jnp.bfloat16)
    hi[...] = mh
    lo[...] = (m - mh.astype(f32)).astype(jnp.bfloat16)


def _edgemlp(Ga, Gb, W, b):
    blk = 2048
    return pl.pallas_call(
        _edgemlp_body,
        grid=(BE // blk,),
        in_specs=[
            pl.BlockSpec((blk, H), lambda i: (i, 0)),
            pl.BlockSpec((blk, H), lambda i: (i, 0)),
            pl.BlockSpec((H, H), lambda i: (0, 0)),
            pl.BlockSpec((1, H), lambda i: (0, 0)),
        ],
        out_specs=[
            pl.BlockSpec((blk, H), lambda i: (i, 0)),
            pl.BlockSpec((blk, H), lambda i: (i, 0)),
        ],
        out_shape=[jax.ShapeDtypeStruct((BE, H), jnp.bfloat16)] * 2,
    )(Ga, Gb, W, b.reshape(1, -1))


_OH_BK = 2048


def _onehot_body(d, o):
    k = pl.program_id(0)
    rows = lax.broadcasted_iota(i32, (N, _OH_BK), 0)
    o[...] = (d[...] == rows).astype(jnp.bfloat16)


def _onehot(dst_row):
    return pl.pallas_call(
        _onehot_body,
        grid=(E // _OH_BK,),
        in_specs=[pl.BlockSpec((1, _OH_BK), lambda k: (0, k))],
        out_specs=pl.BlockSpec((N, _OH_BK), lambda k: (0, k)),
        out_shape=jax.ShapeDtypeStruct((N, E), jnp.bfloat16),
    )(dst_row)


def _segmm_body(oh, hi, lo, o):
    k = pl.program_id(0)

    @pl.when(k == 0)
    def _():
        o[...] = jnp.zeros_like(o)

    ohv = oh[...]
    o[...] += (jnp.dot(ohv, hi[...], preferred_element_type=f32) +
               jnp.dot(ohv, lo[...], preferred_element_type=f32))


def _segmm(OH, mhi, mlo):
    return pl.pallas_call(
        _segmm_body,
        grid=(E // _OH_BK,),
        in_specs=[
            pl.BlockSpec((N, _OH_BK), lambda k: (0, k)),
            pl.BlockSpec((_OH_BK, BH), lambda k: (k, 0)),
            pl.BlockSpec((_OH_BK, BH), lambda k: (k, 0)),
        ],
        out_specs=pl.BlockSpec((N, BH), lambda k: (0, 0)),
        out_shape=jax.ShapeDtypeStruct((N, BH), f32),
    )(OH, mhi, mlo)


def _node_body(h, a0, wut, wub, bu1, wu2, bu2, g, wg, bg, wnt, wnb, bn,
               h2, ps, pd):
    blk = h.shape[0]
    u = _silu(h[...] @ wut[...] + a0[...] @ wub[...] + bu1[...])
    out = u @ wu2[...] + bu2[...]
    film = g[...] @ wg[...] + bg[...]
    filmt = jnp.broadcast_to(film.reshape(1, B, 2 * H),
                             (blk // B, B, 2 * H)).reshape(blk, 2 * H)
    hn = _silu(filmt[:, :H] * out + filmt[:, H:])
    h2[...] = hn
    ps[...] = hn @ wnt[...]
    pd[...] = hn @ wnb[...] + bn[...]


def _node(h, a0, Wut, Wub, bu1, Wu2, bu2, g, Wg, bg, Wnt, Wnb, bn):
    ind = h.shape[1]
    blk = 1024
    full = lambda r, c: pl.BlockSpec((r, c), lambda i: (0, 0))
    rows = lambda c: pl.BlockSpec((blk, c), lambda i: (i, 0))
    return pl.pallas_call(
        _node_body,
        grid=(BN // blk,),
        in_specs=[
            rows(ind), rows(H),
            full(ind, H), full(H, H), full(1, H),
            full(H, H), full(1, H),
            full(B, H), full(H, 2 * H), full(1, 2 * H),
            full(H, H), full(H, H), full(1, H),
        ],
        out_specs=[rows(H), rows(H), rows(H)],
        out_shape=[jax.ShapeDtypeStruct((BN, H), f32)] * 3,
    )(h, a0, Wut, Wub, bu1.reshape(1, -1), Wu2, bu2.reshape(1, -1),
      g, Wg, bg.reshape(1, -1), Wnt, Wnb, bn.reshape(1, -1))


def _erate_body(fa, fb, w, b, o):
    blk = fa.shape[0]
    x = _silu(fa[...] + fb[...])
    v = jnp.sum(x * w[...], axis=1) + b[0, 0]
    v = jnp.maximum(v, 0.0) + jnp.log(1.0 + jnp.exp(-jnp.abs(v)))
    o[...] = v.reshape(blk // B, B)


def _erate(Fa, Fb, w_row, b_sc):
    blk = 2048
    return pl.pallas_call(
        _erate_body,
        grid=(BE // blk,),
        in_specs=[
            pl.BlockSpec((blk, H), lambda i: (i, 0)),
            pl.BlockSpec((blk, H), lambda i: (i, 0)),
            pl.BlockSpec((1, H), lambda i: (0, 0)),
            pl.BlockSpec((1, 1), lambda i: (0, 0)),
        ],
        out_specs=pl.BlockSpec((blk // B, B), lambda i: (i, 0)),
        out_shape=jax.ShapeDtypeStruct((E, B), f32),
    )(Fa, Fb, w_row, b_sc.reshape(1, 1))


_DD_BI, _DD_BJ = 2048, 2048


def _dedup_body(ki, kj, o):
    i = pl.program_id(0)
    j = pl.program_id(1)
    pi = i * _DD_BI + lax.broadcasted_iota(i32, (_DD_BI, 1), 0)
    pj = j * _DD_BJ + lax.broadcasted_iota(i32, (1, _DD_BJ), 1)
    m = (ki[...] == kj[...]) & (pi < pj)
    part = jnp.any(m, axis=1, keepdims=True).astype(i32)

    @pl.when(j == 0)
    def _():
        o[...] = part

    @pl.when(j > 0)
    def _():
        o[...] = o[...] | part


def _dedup(key):
    return pl.pallas_call(
        _dedup_body,
        grid=(E // _DD_BI, E // _DD_BJ),
        in_specs=[
            pl.BlockSpec((_DD_BI, 1), lambda i, j: (i, 0)),
            pl.BlockSpec((1, _DD_BJ), lambda i, j: (0, j)),
        ],
        out_specs=pl.BlockSpec((_DD_BI, 1), lambda i, j: (i, 0)),
        out_shape=jax.ShapeDtypeStruct((E, 1), i32),
    )(key.reshape(E, 1), key.reshape(1, E))


_DG_R = 128


def _diag_body(r, o):
    i = pl.program_id(1)
    v = r[...]
    rs = jnp.sum(v, axis=2)
    rows = i * _DG_R + lax.broadcasted_iota(i32, (1, _DG_R, N), 1)
    cols = lax.broadcasted_iota(i32, (1, _DG_R, N), 2)
    o[...] = jnp.where(rows == cols, -rs[..., None], v)


def _diag(rate):
    return pl.pallas_call(
        _diag_body,
        grid=(B, N // _DG_R),
        in_specs=[pl.BlockSpec((1, _DG_R, N), lambda b, i: (b, i, 0))],
        out_specs=pl.BlockSpec((1, _DG_R, N), lambda b, i: (b, i, 0)),
        out_shape=jax.ShapeDtypeStruct((B, N, N), f32),
    )(rate)


def _sc_mesh():
    return plsc.VectorSubcoreMesh(core_axis_name="c", subcore_axis_name="s")


_CH = 64
_NCH = 8


def _gather2(ptab_s, ptab_d, sidx, didx):
    @functools.partial(
        pl.kernel,
        out_type=(jax.ShapeDtypeStruct((E, BH), f32),
                  jax.ShapeDtypeStruct((E, BH), f32)),
        mesh=_sc_mesh(),
        scratch_types=[
            pltpu.VMEM((_NCH, _CH), i32),
            pltpu.VMEM((_NCH, _CH), i32),
            pltpu.VMEM((_CH, BH), f32),
        ],
    )
    def k(ps, pd, si, di, ga, gb, siv, div, buf):
        c = lax.axis_index("c")
        s = lax.axis_index("s")
        w = c * 16 + s
        pltpu.sync_copy(si.at[pl.ds(w * _NCH, _NCH)], siv)
        pltpu.sync_copy(di.at[pl.ds(w * _NCH, _NCH)], div)

        def body(j, carry):
            base = w * (_NCH * _CH) + j * _CH
            pltpu.sync_copy(ps.at[siv.at[j]], buf)
            pltpu.sync_copy(buf, ga.at[pl.ds(base, _CH)])
            pltpu.sync_copy(pd.at[div.at[j]], buf)
            pltpu.sync_copy(buf, gb.at[pl.ds(base, _CH)])
            return carry

        lax.fori_loop(0, _NCH, body, 0)

    return k(ptab_s, ptab_d, sidx, didx)


def _scatter(vals, fidx, zeros1d):
    @functools.partial(
        pl.kernel,
        out_type=jax.ShapeDtypeStruct((TOT + PAD,), f32),
        mesh=_sc_mesh(),
        scratch_types=[
            pltpu.VMEM((32, 128), i32),
            pltpu.VMEM((65536,), f32),
            pltpu.VMEM((128,), f32),
        ],
    )
    def k(v, fi, z, out, fiv, zbuf, vbuf):
        c = lax.axis_index("c")
        s = lax.axis_index("s")
        pltpu.sync_copy(z, zbuf)

        def zb(kk, carry):
            pltpu.sync_copy(
                zbuf,
                out.at[pl.ds(c * (TOT // 2) + s * (TOT // 32) + kk * 65536,
                             65536)])
            return carry

        lax.fori_loop(0, (TOT // 32) // 65536, zb, 0)
        plsc.subcore_barrier()
        row0 = c * 512 + s * 32
        pltpu.sync_copy(fi.at[pl.ds(row0, 32)], fiv)

        def body(j, carry):
            pltpu.sync_copy(v.at[pl.ds((row0 + j) * 128, 128)], vbuf)
            pltpu.sync_copy(vbuf, out.at[fiv.at[j]])
            return carry

        lax.fori_loop(0, 32, body, 0)

    return k(vals, fidx, zeros1d)


def kernel(mu, t, node_context, global_cond, edge_index, params):
    ge, mp, emlp = params
    (W1, b1), (W2, b2) = ge
    (We1, be1), (We2, be2) = emlp

    src = edge_index[0].astype(i32)
    dst = edge_index[1].astype(i32)
    sidx = src.reshape(E // _CH, _CH)
    didx = dst.reshape(E // _CH, _CH)

    t_exp = jnp.broadcast_to(t, (B, N))
    nf = jnp.concatenate([mu[..., None], t_exp[..., None], node_context],
                         axis=-1)
    h = nf.transpose(1, 0, 2).reshape(BN, 2 + node_context.shape[-1])

    g = _glob_embed(global_cond, W1, b1, W2, b2)
    zeros1d = jnp.zeros((65536,), f32)
    OH = _onehot(dst.reshape(1, E))

    (Wm1, bm1), _, _, _, _ = mp[0]
    ind = h.shape[1]
    Ps, Pd = _proj(h, Wm1[:ind], Wm1[ind:], bm1)

    nlayers = len(mp)
    for li in range(nlayers):
        (Wm1, bm1), (Wm2, bm2), (Wu1, bu1), (Wu2, bu2), (Wg, bg) = mp[li]
        ind = h.shape[1]
        Ga, Gb = _gather2(Ps.reshape(N, BH), Pd.reshape(N, BH), sidx, didx)
        mhi, mlo = _edgemlp(Ga.reshape(BE, H), Gb.reshape(BE, H), Wm2, bm2)
        agg = _segmm(OH, mhi.reshape(E, BH), mlo.reshape(E, BH))
        if li + 1 < nlayers:
            (Wn1, bn1) = mp[li + 1][0]
        else:
            (Wn1, bn1) = (We1, be1)
        h, Ps, Pd = _node(
            h, agg.reshape(BN, H),
            Wu1[:ind], Wu1[ind:], bu1, Wu2, bu2, g, Wg, bg,
            Wn1[:H], Wn1[H:], bn1)

    Fa, Fb = _gather2(Ps.reshape(N, BH), Pd.reshape(N, BH), sidx, didx)
    rates_eb = _erate(Fa.reshape(BE, H), Fb.reshape(BE, H),
                      We2.reshape(1, H), be2)
    rates_be = rates_eb.T.reshape(-1)

    key = src * N + dst
    loser = _dedup(key)[:, 0]
    bidx = jnp.arange(B, dtype=i32)
    flat = bidx[:, None] * (N * N) + key[None, :]
    trash = TOT + (bidx[:, None] // 4) * 64 + jnp.zeros_like(key)[None, :]
    fidx = jnp.where((loser[None, :] != 0), trash, flat).reshape(BE // 128, 128)

    rate_flat = _scatter(rates_be, fidx, zeros1d)
    rate = rate_flat[:TOT].reshape(B, N, N)
    return _diag(rate)


# speedup vs baseline: 4.6048x; 1.8343x over previous
import functools

import jax
import jax.numpy as jnp
from jax import lax
from jax.experimental import pallas as pl
from jax.experimental.pallas import tpu as pltpu
from jax.experimental.pallas import tpu_sc as plsc

B, N, E, H = 8, 1024, 16384, 128
BN = B * N
BE = B * E
BH = B * H
TOT = B * N * N
PAD = 128
f32 = jnp.float32
i32 = jnp.int32


def _silu(x):
    return x * (1.0 / (1.0 + jnp.exp(-x)))


def _glob_embed_body(gc, w1, b1, w2, b2, o):
    x = jnp.maximum(gc[...] @ w1[...] + b1[...], 0.0)
    o[...] = x @ w2[...] + b2[...]


def _glob_embed(gc, W1, b1, W2, b2):
    return pl.pallas_call(
        _glob_embed_body,
        out_shape=jax.ShapeDtypeStruct((B, H), f32),
    )(gc, W1, b1.reshape(1, -1), W2, b2.reshape(1, -1))


def _proj_body(h, wt, wb, bb, ps, pd):
    ind = h.shape[1] // B
    hv = h[...]
    wtv, wbv, bbv = wt[...], wb[...], bb[...]
    ps[...] = jnp.concatenate(
        [hv[:, g * ind:(g + 1) * ind] @ wtv for g in range(B)], axis=1)
    pd[...] = jnp.concatenate(
        [hv[:, g * ind:(g + 1) * ind] @ wbv + bbv for g in range(B)], axis=1)


def _proj(h, Wt, Wb, bb):
    bind = h.shape[1]
    ind = bind // B
    blk = 256
    return pl.pallas_call(
        _proj_body,
        grid=(N // blk,),
        in_specs=[
            pl.BlockSpec((blk, bind), lambda i: (i, 0)),
            pl.BlockSpec((ind, H), lambda i: (0, 0)),
            pl.BlockSpec((ind, H), lambda i: (0, 0)),
            pl.BlockSpec((1, H), lambda i: (0, 0)),
        ],
        out_specs=[
            pl.BlockSpec((blk, BH), lambda i: (i, 0)),
            pl.BlockSpec((blk, BH), lambda i: (i, 0)),
        ],
        out_shape=[jax.ShapeDtypeStruct((N, BH), f32)] * 2,
    )(h, Wt, Wb, bb.reshape(1, -1))


def _edgemlp_body(ga, gb, w, b, hi, lo):
    x = _silu(ga[...] + gb[...])
    wv, bv = w[...], b[...]
    m = jnp.concatenate(
        [x[:, g * H:(g + 1) * H] @ wv + bv for g in range(B)], axis=1)
    mh = m.astype(jnp.bfloat16)
    hi[...] = mh
    lo[...] = (m - mh.astype(f32)).astype(jnp.bfloat16)


def _edgemlp(Ga, Gb, W, b):
    blk = 2048
    return pl.pallas_call(
        _edgemlp_body,
        grid=(E // blk,),
        in_specs=[
            pl.BlockSpec((blk, BH), lambda i: (i, 0)),
            pl.BlockSpec((blk, BH), lambda i: (i, 0)),
            pl.BlockSpec((H, H), lambda i: (0, 0)),
            pl.BlockSpec((1, H), lambda i: (0, 0)),
        ],
        out_specs=[
            pl.BlockSpec((blk, BH), lambda i: (i, 0)),
            pl.BlockSpec((blk, BH), lambda i: (i, 0)),
        ],
        out_shape=[jax.ShapeDtypeStruct((E, BH), jnp.bfloat16)] * 2,
    )(Ga, Gb, W, b.reshape(1, -1))


_OH_BK = 2048


def _segmm_body(d, hi, lo, o):
    k = pl.program_id(0)

    @pl.when(k == 0)
    def _():
        o[...] = jnp.zeros_like(o)

    rows = lax.broadcasted_iota(i32, (N, _OH_BK), 0)
    oh = (d[...] == rows).astype(jnp.bfloat16)
    o[...] += (jnp.dot(oh, hi[...], preferred_element_type=f32) +
               jnp.dot(oh, lo[...], preferred_element_type=f32))


def _segmm(dst_row, mhi, mlo):
    return pl.pallas_call(
        _segmm_body,
        grid=(E // _OH_BK,),
        in_specs=[
            pl.BlockSpec((1, _OH_BK), lambda k: (0, k)),
            pl.BlockSpec((_OH_BK, BH), lambda k: (k, 0)),
            pl.BlockSpec((_OH_BK, BH), lambda k: (k, 0)),
        ],
        out_specs=pl.BlockSpec((N, BH), lambda k: (0, 0)),
        out_shape=jax.ShapeDtypeStruct((N, BH), f32),
    )(dst_row, mhi, mlo)


def _rowsum_body(s, r, l, o):
    k = pl.program_id(0)

    @pl.when(k == 0)
    def _():
        o[...] = jnp.zeros_like(o)

    rows = lax.broadcasted_iota(i32, (N, _OH_BK), 0)
    oh = (s[...] == rows).astype(jnp.bfloat16)
    rv = r[...] * (l[...] == 0).astype(f32)
    hi = rv.astype(jnp.bfloat16)
    lo = (rv - hi.astype(f32)).astype(jnp.bfloat16)
    o[...] += (jnp.dot(oh, hi, preferred_element_type=f32) +
               jnp.dot(oh, lo, preferred_element_type=f32))


def _rowsum(src_row, rates_eb, loser):
    return pl.pallas_call(
        _rowsum_body,
        grid=(E // _OH_BK,),
        in_specs=[
            pl.BlockSpec((1, _OH_BK), lambda k: (0, k)),
            pl.BlockSpec((_OH_BK, B), lambda k: (k, 0)),
            pl.BlockSpec((_OH_BK, 1), lambda k: (k, 0)),
        ],
        out_specs=pl.BlockSpec((N, B), lambda k: (0, 0)),
        out_shape=jax.ShapeDtypeStruct((N, B), f32),
    )(src_row, rates_eb, loser)


def _node_body(h, a0, wut, wub, bu1, wu2, bu2, g, wg, bg, wnt, wnb, bn,
               h2, ps, pd):
    ind = h.shape[1] // B
    hv, av = h[...], a0[...]
    wutv, wubv, bu1v = wut[...], wub[...], bu1[...]
    wu2v, bu2v = wu2[...], bu2[...]
    wntv, wnbv, bnv = wnt[...], wnb[...], bn[...]
    film = g[...] @ wg[...] + bg[...]
    hs, pss, pds = [], [], []
    for b in range(B):
        u = _silu(hv[:, b * ind:(b + 1) * ind] @ wutv +
                  av[:, b * H:(b + 1) * H] @ wubv + bu1v)
        out = u @ wu2v + bu2v
        hn = _silu(film[b:b + 1, :H] * out + film[b:b + 1, H:])
        hs.append(hn)
        pss.append(hn @ wntv)
        pds.append(hn @ wnbv + bnv)
    h2[...] = jnp.concatenate(hs, axis=1)
    ps[...] = jnp.concatenate(pss, axis=1)
    pd[...] = jnp.concatenate(pds, axis=1)


def _node(h, a0, Wut, Wub, bu1, Wu2, bu2, g, Wg, bg, Wnt, Wnb, bn):
    bind = h.shape[1]
    ind = bind // B
    blk = 256
    full = lambda r, c: pl.BlockSpec((r, c), lambda i: (0, 0))
    rows = lambda c: pl.BlockSpec((blk, c), lambda i: (i, 0))
    return pl.pallas_call(
        _node_body,
        grid=(N // blk,),
        in_specs=[
            rows(bind), rows(BH),
            full(ind, H), full(H, H), full(1, H),
            full(H, H), full(1, H),
            full(B, H), full(H, 2 * H), full(1, 2 * H),
            full(H, H), full(H, H), full(1, H),
        ],
        out_specs=[rows(BH), rows(BH), rows(BH)],
        out_shape=[jax.ShapeDtypeStruct((N, BH), f32)] * 3,
    )(h, a0, Wut, Wub, bu1.reshape(1, -1), Wu2, bu2.reshape(1, -1),
      g, Wg, bg.reshape(1, -1), Wnt, Wnb, bn.reshape(1, -1))


def _erate_body(fa, fb, w, b, o):
    x = _silu(fa[...] + fb[...])
    v = x @ w[...] + b[0, 0]
    o[...] = jnp.maximum(v, 0.0) + jnp.log(1.0 + jnp.exp(-jnp.abs(v)))


def _erate(Fa, Fb, Wbig, b_sc):
    blk = 2048
    return pl.pallas_call(
        _erate_body,
        grid=(E // blk,),
        in_specs=[
            pl.BlockSpec((blk, BH), lambda i: (i, 0)),
            pl.BlockSpec((blk, BH), lambda i: (i, 0)),
            pl.BlockSpec((BH, B), lambda i: (0, 0)),
            pl.BlockSpec((1, 1), lambda i: (0, 0)),
        ],
        out_specs=pl.BlockSpec((blk, B), lambda i: (i, 0)),
        out_shape=jax.ShapeDtypeStruct((E, B), f32),
    )(Fa, Fb, Wbig, b_sc.reshape(1, 1))


_DD_BI, _DD_BJ = 2048, 2048


def _dedup_body(ki, kj, o):
    i = pl.program_id(0)
    j = pl.program_id(1)
    pi = i * _DD_BI + lax.broadcasted_iota(i32, (_DD_BI, 1), 0)
    pj = j * _DD_BJ + lax.broadcasted_iota(i32, (1, _DD_BJ), 1)
    m = (ki[...] == kj[...]) & (pi < pj)
    part = jnp.any(m, axis=1, keepdims=True).astype(i32)

    @pl.when(j == 0)
    def _():
        o[...] = part

    @pl.when(j > 0)
    def _():
        o[...] = o[...] | part


def _dedup(key):
    return pl.pallas_call(
        _dedup_body,
        grid=(E // _DD_BI, E // _DD_BJ),
        in_specs=[
            pl.BlockSpec((_DD_BI, 1), lambda i, j: (i, 0)),
            pl.BlockSpec((1, _DD_BJ), lambda i, j: (0, j)),
        ],
        out_specs=pl.BlockSpec((_DD_BI, 1), lambda i, j: (i, 0)),
        out_shape=jax.ShapeDtypeStruct((E, 1), i32),
    )(key.reshape(E, 1), key.reshape(1, E))


def _sc_mesh():
    return plsc.VectorSubcoreMesh(core_axis_name="c", subcore_axis_name="s")


_CH = 64
_NCH = 8


def _gather2(ptab_s, ptab_d, sidx, didx):
    @functools.partial(
        pl.kernel,
        out_type=(jax.ShapeDtypeStruct((E, BH), f32),
                  jax.ShapeDtypeStruct((E, BH), f32)),
        mesh=_sc_mesh(),
        scratch_types=[
            pltpu.VMEM((_NCH, _CH), i32),
            pltpu.VMEM((_NCH, _CH), i32),
            pltpu.VMEM((_CH, BH), f32),
        ],
    )
    def k(ps, pd, si, di, ga, gb, siv, div, buf):
        c = lax.axis_index("c")
        s = lax.axis_index("s")
        w = c * 16 + s
        pltpu.sync_copy(si.at[pl.ds(w * _NCH, _NCH)], siv)
        pltpu.sync_copy(di.at[pl.ds(w * _NCH, _NCH)], div)

        def body(j, carry):
            base = w * (_NCH * _CH) + j * _CH
            pltpu.sync_copy(ps.at[siv.at[j]], buf)
            pltpu.sync_copy(buf, ga.at[pl.ds(base, _CH)])
            pltpu.sync_copy(pd.at[div.at[j]], buf)
            pltpu.sync_copy(buf, gb.at[pl.ds(base, _CH)])
            return carry

        lax.fori_loop(0, _NCH, body, 0)

    return k(ptab_s, ptab_d, sidx, didx)


def _scatter(vals, fidx, dvals, didx, zeros1d):
    @functools.partial(
        pl.kernel,
        out_type=jax.ShapeDtypeStruct((TOT + PAD,), f32),
        mesh=_sc_mesh(),
        scratch_types=[
            pltpu.VMEM((32, 128), i32),
            pltpu.VMEM((65536,), f32),
            pltpu.VMEM((32, 128), f32),
            pltpu.VMEM((2, 128), i32),
            pltpu.VMEM((2, 128), f32),
        ],
    )
    def k(v, fi, dv, di, z, out, fiv, zbuf, vbuf, div, dvb):
        c = lax.axis_index("c")
        s = lax.axis_index("s")
        pltpu.sync_copy(z, zbuf)

        def zb(kk, carry):
            pltpu.sync_copy(
                zbuf,
                out.at[pl.ds(c * (TOT // 2) + s * (TOT // 32) + kk * 65536,
                             65536)])
            return carry

        lax.fori_loop(0, (TOT // 32) // 65536, zb, 0)
        plsc.subcore_barrier()
        row0 = c * 512 + s * 32
        pltpu.sync_copy(fi.at[pl.ds(row0, 32)], fiv)
        pltpu.sync_copy(v.at[pl.ds(row0, 32)], vbuf)

        def body(j, carry):
            pltpu.sync_copy(vbuf.at[j], out.at[fiv.at[j]])
            return carry

        lax.fori_loop(0, 32, body, 0)
        drow0 = c * 32 + s * 2
        pltpu.sync_copy(di.at[pl.ds(drow0, 2)], div)
        pltpu.sync_copy(dv.at[pl.ds(drow0, 2)], dvb)

        def dbody(j, carry):
            pltpu.sync_copy(dvb.at[j], out.at[div.at[j]])
            return carry

        lax.fori_loop(0, 2, dbody, 0)

    return k(vals.reshape(BE // 128, 128), fidx,
             dvals.reshape(BN // 128, 128), didx, zeros1d)


def kernel(mu, t, node_context, global_cond, edge_index, params):
    ge, mp, emlp = params
    (W1, b1), (W2, b2) = ge
    (We1, be1), (We2, be2) = emlp

    src = edge_index[0].astype(i32)
    dst = edge_index[1].astype(i32)
    sidx = src.reshape(E // _CH, _CH)
    didx = dst.reshape(E // _CH, _CH)

    t_exp = jnp.broadcast_to(t, (B, N))
    nf = jnp.concatenate([mu[..., None], t_exp[..., None], node_context],
                         axis=-1)
    h = nf.transpose(1, 0, 2).reshape(N, B * (2 + node_context.shape[-1]))

    g = _glob_embed(global_cond, W1, b1, W2, b2)
    zeros1d = jnp.zeros((65536,), f32)
    dst_row = dst.reshape(1, E)

    (Wm1, bm1), _, _, _, _ = mp[0]
    ind = h.shape[1] // B
    Ps, Pd = _proj(h, Wm1[:ind], Wm1[ind:], bm1)

    nlayers = len(mp)
    for li in range(nlayers):
        (Wm1, bm1), (Wm2, bm2), (Wu1, bu1), (Wu2, bu2), (Wg, bg) = mp[li]
        ind = h.shape[1] // B
        Ga, Gb = _gather2(Ps, Pd, sidx, didx)
        mhi, mlo = _edgemlp(Ga, Gb, Wm2, bm2)
        agg = _segmm(dst_row, mhi, mlo)
        if li + 1 < nlayers:
            (Wn1, bn1) = mp[li + 1][0]
        else:
            (Wn1, bn1) = (We1, be1)
        h, Ps, Pd = _node(
            h, agg,
            Wu1[:ind], Wu1[ind:], bu1, Wu2, bu2, g, Wg, bg,
            Wn1[:H], Wn1[H:], bn1)

    Fa, Fb = _gather2(Ps, Pd, sidx, didx)
    Wbig = jnp.kron(jnp.eye(B, dtype=f32), We2)
    rates_eb = _erate(Fa, Fb, Wbig, be2)
    rates_be = rates_eb.T.reshape(-1)

    key = src * N + dst
    loser = _dedup(key)
    rs = _rowsum(src.reshape(1, E), rates_eb, loser)
    dvals = (-rs).T.reshape(-1)

    bidx = jnp.arange(B, dtype=i32)
    redirect = (loser[:, 0] != 0) | (src == dst)
    flat = bidx[:, None] * (N * N) + key[None, :]
    fidx = jnp.where(redirect[None, :], TOT, flat).reshape(BE // 128, 128)
    diag_pos = jnp.arange(N, dtype=i32) * (N + 1)
    didx = (bidx[:, None] * (N * N) + diag_pos[None, :]).reshape(BN // 128, 128)

    rate_flat = _scatter(rates_be, fidx, dvals, didx, zeros1d)
    return rate_flat[:TOT].reshape(B, N, N)


# speedup vs baseline: 4.6161x; 1.0024x over previous
import functools

import jax
import jax.numpy as jnp
from jax import lax
from jax.experimental import pallas as pl
from jax.experimental.pallas import tpu as pltpu
from jax.experimental.pallas import tpu_sc as plsc

B, N, E, H = 8, 1024, 16384, 128
BN = B * N
BE = B * E
BH = B * H
TOT = B * N * N
PAD = 128
f32 = jnp.float32
i32 = jnp.int32


def _silu(x):
    return x * (1.0 / (1.0 + jnp.exp(-x)))


def _glob_embed_body(gc, w1, b1, w2, b2, o):
    x = jnp.maximum(gc[...] @ w1[...] + b1[...], 0.0)
    o[...] = x @ w2[...] + b2[...]


def _glob_embed(gc, W1, b1, W2, b2):
    return pl.pallas_call(
        _glob_embed_body,
        out_shape=jax.ShapeDtypeStruct((B, H), f32),
    )(gc, W1, b1.reshape(1, -1), W2, b2.reshape(1, -1))


def _proj_body(h, wt, wb, bb, ps, pd):
    ind = h.shape[1] // B
    hv = h[...]
    wtv, wbv, bbv = wt[...], wb[...], bb[...]
    ps[...] = jnp.concatenate(
        [hv[:, g * ind:(g + 1) * ind] @ wtv for g in range(B)], axis=1)
    pd[...] = jnp.concatenate(
        [hv[:, g * ind:(g + 1) * ind] @ wbv + bbv for g in range(B)], axis=1)


def _proj(h, Wt, Wb, bb):
    bind = h.shape[1]
    ind = bind // B
    blk = 256
    return pl.pallas_call(
        _proj_body,
        grid=(N // blk,),
        in_specs=[
            pl.BlockSpec((blk, bind), lambda i: (i, 0)),
            pl.BlockSpec((ind, H), lambda i: (0, 0)),
            pl.BlockSpec((ind, H), lambda i: (0, 0)),
            pl.BlockSpec((1, H), lambda i: (0, 0)),
        ],
        out_specs=[
            pl.BlockSpec((blk, BH), lambda i: (i, 0)),
            pl.BlockSpec((blk, BH), lambda i: (i, 0)),
        ],
        out_shape=[jax.ShapeDtypeStruct((N, BH), f32)] * 2,
    )(h, Wt, Wb, bb.reshape(1, -1))


def _edgemlp_body(ga, gb, w, b, hi, lo):
    x = _silu(ga[...] + gb[...])
    wv, bv = w[...], b[...]
    m = jnp.concatenate(
        [x[:, g * H:(g + 1) * H] @ wv + bv for g in range(B)], axis=1)
    mh = m.astype(jnp.bfloat16)
    hi[...] = mh
    lo[...] = (m - mh.astype(f32)).astype(jnp.bfloat16)


def _edgemlp(Ga, Gb, W, b):
    ne = Ga.shape[0]
    blk = 2048
    return pl.pallas_call(
        _edgemlp_body,
        grid=(ne // blk,),
        in_specs=[
            pl.BlockSpec((blk, BH), lambda i: (i, 0)),
            pl.BlockSpec((blk, BH), lambda i: (i, 0)),
            pl.BlockSpec((H, H), lambda i: (0, 0)),
            pl.BlockSpec((1, H), lambda i: (0, 0)),
        ],
        out_specs=[
            pl.BlockSpec((blk, BH), lambda i: (i, 0)),
            pl.BlockSpec((blk, BH), lambda i: (i, 0)),
        ],
        out_shape=[jax.ShapeDtypeStruct((ne, BH), jnp.bfloat16)] * 2,
    )(Ga, Gb, W, b.reshape(1, -1))


_OH_BK = 2048


def _segmm_body(d, hi, lo, o):
    k = pl.program_id(0)

    @pl.when(k == 0)
    def _():
        o[...] = jnp.zeros_like(o)

    rows = lax.broadcasted_iota(i32, (N, _OH_BK), 0)
    oh = (d[...] == rows).astype(jnp.bfloat16)
    o[...] += (jnp.dot(oh, hi[...], preferred_element_type=f32) +
               jnp.dot(oh, lo[...], preferred_element_type=f32))


def _segmm(dst_row, mhi, mlo):
    ne = dst_row.shape[1]
    return pl.pallas_call(
        _segmm_body,
        grid=(ne // _OH_BK,),
        in_specs=[
            pl.BlockSpec((1, _OH_BK), lambda k: (0, k)),
            pl.BlockSpec((_OH_BK, BH), lambda k: (k, 0)),
            pl.BlockSpec((_OH_BK, BH), lambda k: (k, 0)),
        ],
        out_specs=pl.BlockSpec((N, BH), lambda k: (0, 0)),
        out_shape=jax.ShapeDtypeStruct((N, BH), f32),
    )(dst_row, mhi, mlo)


def _rowsum_body(s, r, l, o):
    k = pl.program_id(0)

    @pl.when(k == 0)
    def _():
        o[...] = jnp.zeros_like(o)

    rows = lax.broadcasted_iota(i32, (N, _OH_BK), 0)
    oh = (s[...] == rows).astype(jnp.bfloat16)
    rv = r[...] * (l[...] == 0).astype(f32)
    hi = rv.astype(jnp.bfloat16)
    lo = (rv - hi.astype(f32)).astype(jnp.bfloat16)
    o[...] += (jnp.dot(oh, hi, preferred_element_type=f32) +
               jnp.dot(oh, lo, preferred_element_type=f32))


def _rowsum(src_row, rates_eb, loser):
    return pl.pallas_call(
        _rowsum_body,
        grid=(E // _OH_BK,),
        in_specs=[
            pl.BlockSpec((1, _OH_BK), lambda k: (0, k)),
            pl.BlockSpec((_OH_BK, B), lambda k: (k, 0)),
            pl.BlockSpec((_OH_BK, 1), lambda k: (k, 0)),
        ],
        out_specs=pl.BlockSpec((N, B), lambda k: (0, 0)),
        out_shape=jax.ShapeDtypeStruct((N, B), f32),
    )(src_row, rates_eb, loser)


def _node_body(h, a0, a1, wut, wub, bu1, wu2, bu2, g, wg, bg, wnt, wnb, bn,
               h2, ps, pd):
    ind = h.shape[1] // B
    hv, av = h[...], a0[...] + a1[...]
    wutv, wubv, bu1v = wut[...], wub[...], bu1[...]
    wu2v, bu2v = wu2[...], bu2[...]
    wntv, wnbv, bnv = wnt[...], wnb[...], bn[...]
    film = g[...] @ wg[...] + bg[...]
    hs, pss, pds = [], [], []
    for b in range(B):
        u = _silu(hv[:, b * ind:(b + 1) * ind] @ wutv +
                  av[:, b * H:(b + 1) * H] @ wubv + bu1v)
        out = u @ wu2v + bu2v
        hn = _silu(film[b:b + 1, :H] * out + film[b:b + 1, H:])
        hs.append(hn)
        pss.append(hn @ wntv)
        pds.append(hn @ wnbv + bnv)
    h2[...] = jnp.concatenate(hs, axis=1)
    ps[...] = jnp.concatenate(pss, axis=1)
    pd[...] = jnp.concatenate(pds, axis=1)


def _node(h, a0, a1, Wut, Wub, bu1, Wu2, bu2, g, Wg, bg, Wnt, Wnb, bn):
    bind = h.shape[1]
    ind = bind // B
    blk = 256
    full = lambda r, c: pl.BlockSpec((r, c), lambda i: (0, 0))
    rows = lambda c: pl.BlockSpec((blk, c), lambda i: (i, 0))
    return pl.pallas_call(
        _node_body,
        grid=(N // blk,),
        in_specs=[
            rows(bind), rows(BH), rows(BH),
            full(ind, H), full(H, H), full(1, H),
            full(H, H), full(1, H),
            full(B, H), full(H, 2 * H), full(1, 2 * H),
            full(H, H), full(H, H), full(1, H),
        ],
        out_specs=[rows(BH), rows(BH), rows(BH)],
        out_shape=[jax.ShapeDtypeStruct((N, BH), f32)] * 3,
    )(h, a0, a1, Wut, Wub, bu1.reshape(1, -1), Wu2, bu2.reshape(1, -1),
      g, Wg, bg.reshape(1, -1), Wnt, Wnb, bn.reshape(1, -1))


def _erate_body(fa, fb, w, b, o):
    x = _silu(fa[...] + fb[...])
    v = x @ w[...] + b[0, 0]
    o[...] = jnp.maximum(v, 0.0) + jnp.log(1.0 + jnp.exp(-jnp.abs(v)))


def _erate(Fa, Fb, Wbig, b_sc):
    ne = Fa.shape[0]
    blk = 2048
    return pl.pallas_call(
        _erate_body,
        grid=(ne // blk,),
        in_specs=[
            pl.BlockSpec((blk, BH), lambda i: (i, 0)),
            pl.BlockSpec((blk, BH), lambda i: (i, 0)),
            pl.BlockSpec((BH, B), lambda i: (0, 0)),
            pl.BlockSpec((1, 1), lambda i: (0, 0)),
        ],
        out_specs=pl.BlockSpec((blk, B), lambda i: (i, 0)),
        out_shape=jax.ShapeDtypeStruct((ne, B), f32),
    )(Fa, Fb, Wbig, b_sc.reshape(1, 1))


_DD_BI, _DD_BJ = 2048, 2048


def _dedup_body(ki, kj, o):
    i = pl.program_id(0)
    j = pl.program_id(1)
    pi = i * _DD_BI + lax.broadcasted_iota(i32, (_DD_BI, 1), 0)
    pj = j * _DD_BJ + lax.broadcasted_iota(i32, (1, _DD_BJ), 1)
    m = (ki[...] == kj[...]) & (pi < pj)
    part = jnp.any(m, axis=1, keepdims=True).astype(i32)

    @pl.when(j == 0)
    def _():
        o[...] = part

    @pl.when(j > 0)
    def _():
        o[...] = o[...] | part


def _dedup(key):
    return pl.pallas_call(
        _dedup_body,
        grid=(E // _DD_BI, E // _DD_BJ),
        in_specs=[
            pl.BlockSpec((_DD_BI, 1), lambda i, j: (i, 0)),
            pl.BlockSpec((1, _DD_BJ), lambda i, j: (0, j)),
        ],
        out_specs=pl.BlockSpec((_DD_BI, 1), lambda i, j: (i, 0)),
        out_shape=jax.ShapeDtypeStruct((E, 1), i32),
    )(key.reshape(E, 1), key.reshape(1, E))


def _sc_mesh():
    return plsc.VectorSubcoreMesh(core_axis_name="c", subcore_axis_name="s")


_CH = 64
_NCH = 8


def _gather2(ptab_s, ptab_d, sidx, didx):
    ne = sidx.shape[0] * _CH
    nch = ne // (32 * _CH)

    @functools.partial(
        pl.kernel,
        out_type=(jax.ShapeDtypeStruct((ne, BH), f32),
                  jax.ShapeDtypeStruct((ne, BH), f32)),
        mesh=_sc_mesh(),
        scratch_types=[
            pltpu.VMEM((nch, _CH), i32),
            pltpu.VMEM((nch, _CH), i32),
            pltpu.VMEM((_CH, BH), f32),
        ],
    )
    def k(ps, pd, si, di, ga, gb, siv, div, buf):
        c = lax.axis_index("c")
        s = lax.axis_index("s")
        w = c * 16 + s
        pltpu.sync_copy(si.at[pl.ds(w * nch, nch)], siv)
        pltpu.sync_copy(di.at[pl.ds(w * nch, nch)], div)

        def body(j, carry):
            base = w * (nch * _CH) + j * _CH
            pltpu.sync_copy(ps.at[siv.at[j]], buf)
            pltpu.sync_copy(buf, ga.at[pl.ds(base, _CH)])
            pltpu.sync_copy(pd.at[div.at[j]], buf)
            pltpu.sync_copy(buf, gb.at[pl.ds(base, _CH)])
            return carry

        lax.fori_loop(0, nch, body, 0)

    return k(ptab_s, ptab_d, sidx, didx)


def _scatter(vals, fidx, dvals, didx, zeros1d):
    @functools.partial(
        pl.kernel,
        out_type=jax.ShapeDtypeStruct((TOT + PAD,), f32),
        mesh=_sc_mesh(),
        scratch_types=[
            pltpu.VMEM((32, 128), i32),
            pltpu.VMEM((65536,), f32),
            pltpu.VMEM((32, 128), f32),
            pltpu.VMEM((2, 128), i32),
            pltpu.VMEM((2, 128), f32),
        ],
    )
    def k(v, fi, dv, di, z, out, fiv, zbuf, vbuf, div, dvb):
        c = lax.axis_index("c")
        s = lax.axis_index("s")
        pltpu.sync_copy(z, zbuf)

        def zb(kk, carry):
            pltpu.sync_copy(
                zbuf,
                out.at[pl.ds(c * (TOT // 2) + s * (TOT // 32) + kk * 65536,
                             65536)])
            return carry

        lax.fori_loop(0, (TOT // 32) // 65536, zb, 0)
        plsc.subcore_barrier()
        row0 = c * 512 + s * 32
        pltpu.sync_copy(fi.at[pl.ds(row0, 32)], fiv)
        pltpu.sync_copy(v.at[pl.ds(row0, 32)], vbuf)

        def body(j, carry):
            pltpu.sync_copy(vbuf.at[j], out.at[fiv.at[j]])
            return carry

        lax.fori_loop(0, 32, body, 0)
        drow0 = c * 32 + s * 2
        pltpu.sync_copy(di.at[pl.ds(drow0, 2)], div)
        pltpu.sync_copy(dv.at[pl.ds(drow0, 2)], dvb)

        def dbody(j, carry):
            pltpu.sync_copy(dvb.at[j], out.at[div.at[j]])
            return carry

        lax.fori_loop(0, 2, dbody, 0)

    return k(vals.reshape(BE // 128, 128), fidx,
             dvals.reshape(BN // 128, 128), didx, zeros1d)


def kernel(mu, t, node_context, global_cond, edge_index, params):
    ge, mp, emlp = params
    (W1, b1), (W2, b2) = ge
    (We1, be1), (We2, be2) = emlp

    src = edge_index[0].astype(i32)
    dst = edge_index[1].astype(i32)
    sidx = src.reshape(E // _CH, _CH)
    didx = dst.reshape(E // _CH, _CH)
    hrow = E // (2 * _CH)
    sidx1, sidx2 = sidx[:hrow], sidx[hrow:]
    didx1, didx2 = didx[:hrow], didx[hrow:]

    t_exp = jnp.broadcast_to(t, (B, N))
    nf = jnp.concatenate([mu[..., None], t_exp[..., None], node_context],
                         axis=-1)
    h = nf.transpose(1, 0, 2).reshape(N, B * (2 + node_context.shape[-1]))

    g = _glob_embed(global_cond, W1, b1, W2, b2)
    zeros1d = jnp.zeros((65536,), f32)
    dst1 = dst[:E // 2].reshape(1, E // 2)
    dst2 = dst[E // 2:].reshape(1, E // 2)

    (Wm1, bm1), _, _, _, _ = mp[0]
    ind = h.shape[1] // B
    Ps, Pd = _proj(h, Wm1[:ind], Wm1[ind:], bm1)

    nlayers = len(mp)
    for li in range(nlayers):
        (Wm1, bm1), (Wm2, bm2), (Wu1, bu1), (Wu2, bu2), (Wg, bg) = mp[li]
        ind = h.shape[1] // B
        Ga1, Gb1 = _gather2(Ps, Pd, sidx1, didx1)
        Ga2, Gb2 = _gather2(Ps, Pd, sidx2, didx2)
        mhi1, mlo1 = _edgemlp(Ga1, Gb1, Wm2, bm2)
        agg1 = _segmm(dst1, mhi1, mlo1)
        mhi2, mlo2 = _edgemlp(Ga2, Gb2, Wm2, bm2)
        agg2 = _segmm(dst2, mhi2, mlo2)
        if li + 1 < nlayers:
            (Wn1, bn1) = mp[li + 1][0]
        else:
            (Wn1, bn1) = (We1, be1)
        h, Ps, Pd = _node(
            h, agg1, agg2,
            Wu1[:ind], Wu1[ind:], bu1, Wu2, bu2, g, Wg, bg,
            Wn1[:H], Wn1[H:], bn1)

    Fa1, Fb1 = _gather2(Ps, Pd, sidx1, didx1)
    Fa2, Fb2 = _gather2(Ps, Pd, sidx2, didx2)
    Wbig = jnp.kron(jnp.eye(B, dtype=f32), We2)
    rates_eb = jnp.concatenate(
        [_erate(Fa1, Fb1, Wbig, be2), _erate(Fa2, Fb2, Wbig, be2)], axis=0)
    rates_be = rates_eb.T.reshape(-1)

    key = src * N + dst
    loser = _dedup(key)
    rs = _rowsum(src.reshape(1, E), rates_eb, loser)
    dvals = (-rs).T.reshape(-1)

    bidx = jnp.arange(B, dtype=i32)
    redirect = (loser[:, 0] != 0) | (src == dst)
    flat = bidx[:, None] * (N * N) + key[None, :]
    fidx = jnp.where(redirect[None, :], TOT, flat).reshape(BE // 128, 128)
    diag_pos = jnp.arange(N, dtype=i32) * (N + 1)
    didx = (bidx[:, None] * (N * N) + diag_pos[None, :]).reshape(BN // 128, 128)

    rate_flat = _scatter(rates_be, fidx, dvals, didx, zeros1d)
    return rate_flat[:TOT].reshape(B, N, N)


# speedup vs baseline: 4.6209x; 1.0010x over previous
import functools

import jax
import jax.numpy as jnp
from jax import lax
from jax.experimental import pallas as pl
from jax.experimental.pallas import tpu as pltpu
from jax.experimental.pallas import tpu_sc as plsc

B, N, E, H = 8, 1024, 16384, 128
BN = B * N
BE = B * E
BH = B * H
TOT = B * N * N
PAD = 128
f32 = jnp.float32
i32 = jnp.int32


def _silu(x):
    return x * (1.0 / (1.0 + jnp.exp(-x)))


def _glob_embed_body(gc, w1, b1, w2, b2, o):
    x = jnp.maximum(gc[...] @ w1[...] + b1[...], 0.0)
    o[...] = x @ w2[...] + b2[...]


def _glob_embed(gc, W1, b1, W2, b2):
    return pl.pallas_call(
        _glob_embed_body,
        out_shape=jax.ShapeDtypeStruct((B, H), f32),
    )(gc, W1, b1.reshape(1, -1), W2, b2.reshape(1, -1))


def _proj_body(h, wt, wb, bb, ps, pd):
    ind = h.shape[1] // B
    hv = h[...]
    wtv, wbv, bbv = wt[...], wb[...], bb[...]
    ps[...] = jnp.concatenate(
        [hv[:, g * ind:(g + 1) * ind] @ wtv for g in range(B)], axis=1)
    pd[...] = jnp.concatenate(
        [hv[:, g * ind:(g + 1) * ind] @ wbv + bbv for g in range(B)], axis=1)


def _proj(h, Wt, Wb, bb):
    bind = h.shape[1]
    ind = bind // B
    blk = 256
    return pl.pallas_call(
        _proj_body,
        grid=(N // blk,),
        in_specs=[
            pl.BlockSpec((blk, bind), lambda i: (i, 0)),
            pl.BlockSpec((ind, H), lambda i: (0, 0)),
            pl.BlockSpec((ind, H), lambda i: (0, 0)),
            pl.BlockSpec((1, H), lambda i: (0, 0)),
        ],
        out_specs=[
            pl.BlockSpec((blk, BH), lambda i: (i, 0)),
            pl.BlockSpec((blk, BH), lambda i: (i, 0)),
        ],
        out_shape=[jax.ShapeDtypeStruct((N, BH), f32)] * 2,
    )(h, Wt, Wb, bb.reshape(1, -1))


def _edgemlp_body(ga, gb, w, b, hi, lo):
    x = _silu(ga[...] + gb[...])
    wv, bv = w[...], b[...]
    m = jnp.concatenate(
        [x[:, g * H:(g + 1) * H] @ wv + bv for g in range(B)], axis=1)
    mh = m.astype(jnp.bfloat16)
    hi[...] = mh
    lo[...] = (m - mh.astype(f32)).astype(jnp.bfloat16)


def _edgemlp(Ga, Gb, W, b):
    ne = Ga.shape[0]
    blk = 2048
    return pl.pallas_call(
        _edgemlp_body,
        grid=(ne // blk,),
        in_specs=[
            pl.BlockSpec((blk, BH), lambda i: (i, 0)),
            pl.BlockSpec((blk, BH), lambda i: (i, 0)),
            pl.BlockSpec((H, H), lambda i: (0, 0)),
            pl.BlockSpec((1, H), lambda i: (0, 0)),
        ],
        out_specs=[
            pl.BlockSpec((blk, BH), lambda i: (i, 0)),
            pl.BlockSpec((blk, BH), lambda i: (i, 0)),
        ],
        out_shape=[jax.ShapeDtypeStruct((ne, BH), jnp.bfloat16)] * 2,
    )(Ga, Gb, W, b.reshape(1, -1))


_OH_BK = 2048


def _segmm_body(d, hi, lo, o):
    k = pl.program_id(0)

    @pl.when(k == 0)
    def _():
        o[...] = jnp.zeros_like(o)

    rows = lax.broadcasted_iota(i32, (N, _OH_BK), 0)
    oh = (d[...] == rows).astype(jnp.bfloat16)
    o[...] += (jnp.dot(oh, hi[...], preferred_element_type=f32) +
               jnp.dot(oh, lo[...], preferred_element_type=f32))


def _segmm(dst_row, mhi, mlo):
    ne = dst_row.shape[1]
    return pl.pallas_call(
        _segmm_body,
        grid=(ne // _OH_BK,),
        in_specs=[
            pl.BlockSpec((1, _OH_BK), lambda k: (0, k)),
            pl.BlockSpec((_OH_BK, BH), lambda k: (k, 0)),
            pl.BlockSpec((_OH_BK, BH), lambda k: (k, 0)),
        ],
        out_specs=pl.BlockSpec((N, BH), lambda k: (0, 0)),
        out_shape=jax.ShapeDtypeStruct((N, BH), f32),
    )(dst_row, mhi, mlo)


def _rowsum_body(s, r, l, o):
    k = pl.program_id(0)

    @pl.when(k == 0)
    def _():
        o[...] = jnp.zeros_like(o)

    rows = lax.broadcasted_iota(i32, (N, _OH_BK), 0)
    oh = (s[...] == rows).astype(jnp.bfloat16)
    rv = r[...] * (l[...] == 0).astype(f32)
    hi = rv.astype(jnp.bfloat16)
    lo = (rv - hi.astype(f32)).astype(jnp.bfloat16)
    o[...] += (jnp.dot(oh, hi, preferred_element_type=f32) +
               jnp.dot(oh, lo, preferred_element_type=f32))


def _rowsum(src_row, rates_eb, loser):
    return pl.pallas_call(
        _rowsum_body,
        grid=(E // _OH_BK,),
        in_specs=[
            pl.BlockSpec((1, _OH_BK), lambda k: (0, k)),
            pl.BlockSpec((_OH_BK, B), lambda k: (k, 0)),
            pl.BlockSpec((_OH_BK, 1), lambda k: (k, 0)),
        ],
        out_specs=pl.BlockSpec((N, B), lambda k: (0, 0)),
        out_shape=jax.ShapeDtypeStruct((N, B), f32),
    )(src_row, rates_eb, loser)


def _node_body(h, a0, a1, wut, wub, bu1, wu2, bu2, g, wg, bg, wnt, wnb, bn,
               h2, ps, pd):
    ind = h.shape[1] // B
    hv, av = h[...], a0[...] + a1[...]
    wutv, wubv, bu1v = wut[...], wub[...], bu1[...]
    wu2v, bu2v = wu2[...], bu2[...]
    wntv, wnbv, bnv = wnt[...], wnb[...], bn[...]
    film = g[...] @ wg[...] + bg[...]
    hs, pss, pds = [], [], []
    for b in range(B):
        u = _silu(hv[:, b * ind:(b + 1) * ind] @ wutv +
                  av[:, b * H:(b + 1) * H] @ wubv + bu1v)
        out = u @ wu2v + bu2v
        hn = _silu(film[b:b + 1, :H] * out + film[b:b + 1, H:])
        hs.append(hn)
        pss.append(hn @ wntv)
        pds.append(hn @ wnbv + bnv)
    h2[...] = jnp.concatenate(hs, axis=1)
    ps[...] = jnp.concatenate(pss, axis=1)
    pd[...] = jnp.concatenate(pds, axis=1)


def _node(h, a0, a1, Wut, Wub, bu1, Wu2, bu2, g, Wg, bg, Wnt, Wnb, bn):
    bind = h.shape[1]
    ind = bind // B
    blk = 256
    full = lambda r, c: pl.BlockSpec((r, c), lambda i: (0, 0))
    rows = lambda c: pl.BlockSpec((blk, c), lambda i: (i, 0))
    return pl.pallas_call(
        _node_body,
        grid=(N // blk,),
        in_specs=[
            rows(bind), rows(BH), rows(BH),
            full(ind, H), full(H, H), full(1, H),
            full(H, H), full(1, H),
            full(B, H), full(H, 2 * H), full(1, 2 * H),
            full(H, H), full(H, H), full(1, H),
        ],
        out_specs=[rows(BH), rows(BH), rows(BH)],
        out_shape=[jax.ShapeDtypeStruct((N, BH), f32)] * 3,
    )(h, a0, a1, Wut, Wub, bu1.reshape(1, -1), Wu2, bu2.reshape(1, -1),
      g, Wg, bg.reshape(1, -1), Wnt, Wnb, bn.reshape(1, -1))


def _erate_body(fa, fb, w, b, o):
    x = _silu(fa[...] + fb[...])
    v = x @ w[...] + b[0, 0]
    o[...] = jnp.maximum(v, 0.0) + jnp.log(1.0 + jnp.exp(-jnp.abs(v)))


def _erate(Fa, Fb, Wbig, b_sc):
    ne = Fa.shape[0]
    blk = 2048
    return pl.pallas_call(
        _erate_body,
        grid=(ne // blk,),
        in_specs=[
            pl.BlockSpec((blk, BH), lambda i: (i, 0)),
            pl.BlockSpec((blk, BH), lambda i: (i, 0)),
            pl.BlockSpec((BH, B), lambda i: (0, 0)),
            pl.BlockSpec((1, 1), lambda i: (0, 0)),
        ],
        out_specs=pl.BlockSpec((blk, B), lambda i: (i, 0)),
        out_shape=jax.ShapeDtypeStruct((ne, B), f32),
    )(Fa, Fb, Wbig, b_sc.reshape(1, 1))


_DD_BI, _DD_BJ = 2048, 2048


def _dedup_body(ki, kj, o):
    i = pl.program_id(0)
    j = pl.program_id(1)
    pi = i * _DD_BI + lax.broadcasted_iota(i32, (_DD_BI, 1), 0)
    pj = j * _DD_BJ + lax.broadcasted_iota(i32, (1, _DD_BJ), 1)
    m = (ki[...] == kj[...]) & (pi < pj)
    part = jnp.any(m, axis=1, keepdims=True).astype(i32)

    @pl.when(j == 0)
    def _():
        o[...] = part

    @pl.when(j > 0)
    def _():
        o[...] = o[...] | part


def _dedup(key):
    return pl.pallas_call(
        _dedup_body,
        grid=(E // _DD_BI, E // _DD_BJ),
        in_specs=[
            pl.BlockSpec((_DD_BI, 1), lambda i, j: (i, 0)),
            pl.BlockSpec((1, _DD_BJ), lambda i, j: (0, j)),
        ],
        out_specs=pl.BlockSpec((_DD_BI, 1), lambda i, j: (i, 0)),
        out_shape=jax.ShapeDtypeStruct((E, 1), i32),
    )(key.reshape(E, 1), key.reshape(1, E))


def _sc_mesh():
    return plsc.VectorSubcoreMesh(core_axis_name="c", subcore_axis_name="s")


_CH = 64
_NCH = 8


def _gather2(ptab_s, ptab_d, sidx, didx):
    ne = sidx.shape[0] * _CH
    nch = ne // (32 * _CH)

    @functools.partial(
        pl.kernel,
        out_type=(jax.ShapeDtypeStruct((ne, BH), f32),
                  jax.ShapeDtypeStruct((ne, BH), f32)),
        mesh=_sc_mesh(),
        scratch_types=[
            pltpu.VMEM((nch, _CH), i32),
            pltpu.VMEM((nch, _CH), i32),
            pltpu.VMEM((_CH, BH), f32),
        ],
    )
    def k(ps, pd, si, di, ga, gb, siv, div, buf):
        c = lax.axis_index("c")
        s = lax.axis_index("s")
        w = c * 16 + s
        pltpu.sync_copy(si.at[pl.ds(w * nch, nch)], siv)
        pltpu.sync_copy(di.at[pl.ds(w * nch, nch)], div)

        def body(j, carry):
            base = w * (nch * _CH) + j * _CH
            pltpu.sync_copy(ps.at[siv.at[j]], buf)
            pltpu.sync_copy(buf, ga.at[pl.ds(base, _CH)])
            pltpu.sync_copy(pd.at[div.at[j]], buf)
            pltpu.sync_copy(buf, gb.at[pl.ds(base, _CH)])
            return carry

        lax.fori_loop(0, nch, body, 0)

    return k(ptab_s, ptab_d, sidx, didx)


def _scatter(vals, fidx, dvals, didx, zeros1d):
    @functools.partial(
        pl.kernel,
        out_type=jax.ShapeDtypeStruct((TOT + PAD,), f32),
        mesh=_sc_mesh(),
        scratch_types=[
            pltpu.VMEM((32, 128), i32),
            pltpu.VMEM((65536,), f32),
            pltpu.VMEM((32, 128), f32),
            pltpu.VMEM((2, 128), i32),
            pltpu.VMEM((2, 128), f32),
            pltpu.SemaphoreType.DMA,
        ],
    )
    def k(v, fi, dv, di, z, out, fiv, zbuf, vbuf, div, dvb, sem):
        c = lax.axis_index("c")
        s = lax.axis_index("s")
        pltpu.sync_copy(z, zbuf)
        base0 = c * (TOT // 2) + s * (TOT // 32)
        zh = [pltpu.async_copy(zbuf, out.at[pl.ds(base0 + kk * 65536, 65536)],
                               sem)
              for kk in range((TOT // 32) // 65536)]
        for hh in zh:
            hh.wait()
        plsc.subcore_barrier()
        row0 = c * 512 + s * 32
        pltpu.sync_copy(fi.at[pl.ds(row0, 32)], fiv)
        pltpu.sync_copy(v.at[pl.ds(row0, 32)], vbuf)

        def body(it, carry):
            hs = [pltpu.async_copy(vbuf.at[it * 16 + j],
                                   out.at[fiv.at[it * 16 + j]], sem)
                  for j in range(16)]
            for hh in hs:
                hh.wait()
            return carry

        lax.fori_loop(0, 2, body, 0)
        drow0 = c * 32 + s * 2
        pltpu.sync_copy(di.at[pl.ds(drow0, 2)], div)
        pltpu.sync_copy(dv.at[pl.ds(drow0, 2)], dvb)

        def dbody(j, carry):
            pltpu.sync_copy(dvb.at[j], out.at[div.at[j]])
            return carry

        lax.fori_loop(0, 2, dbody, 0)

    return k(vals.reshape(BE // 128, 128), fidx,
             dvals.reshape(BN // 128, 128), didx, zeros1d)


def kernel(mu, t, node_context, global_cond, edge_index, params):
    ge, mp, emlp = params
    (W1, b1), (W2, b2) = ge
    (We1, be1), (We2, be2) = emlp

    src = edge_index[0].astype(i32)
    dst = edge_index[1].astype(i32)
    sidx = src.reshape(E // _CH, _CH)
    didx = dst.reshape(E // _CH, _CH)
    hrow = E // (2 * _CH)
    sidx1, sidx2 = sidx[:hrow], sidx[hrow:]
    didx1, didx2 = didx[:hrow], didx[hrow:]

    t_exp = jnp.broadcast_to(t, (B, N))
    nf = jnp.concatenate([mu[..., None], t_exp[..., None], node_context],
                         axis=-1)
    h = nf.transpose(1, 0, 2).reshape(N, B * (2 + node_context.shape[-1]))

    g = _glob_embed(global_cond, W1, b1, W2, b2)
    zeros1d = jnp.zeros((65536,), f32)
    dst1 = dst[:E // 2].reshape(1, E // 2)
    dst2 = dst[E // 2:].reshape(1, E // 2)

    (Wm1, bm1), _, _, _, _ = mp[0]
    ind = h.shape[1] // B
    Ps, Pd = _proj(h, Wm1[:ind], Wm1[ind:], bm1)

    nlayers = len(mp)
    for li in range(nlayers):
        (Wm1, bm1), (Wm2, bm2), (Wu1, bu1), (Wu2, bu2), (Wg, bg) = mp[li]
        ind = h.shape[1] // B
        Ga1, Gb1 = _gather2(Ps, Pd, sidx1, didx1)
        Ga2, Gb2 = _gather2(Ps, Pd, sidx2, didx2)
        mhi1, mlo1 = _edgemlp(Ga1, Gb1, Wm2, bm2)
        agg1 = _segmm(dst1, mhi1, mlo1)
        mhi2, mlo2 = _edgemlp(Ga2, Gb2, Wm2, bm2)
        agg2 = _segmm(dst2, mhi2, mlo2)
        if li + 1 < nlayers:
            (Wn1, bn1) = mp[li + 1][0]
        else:
            (Wn1, bn1) = (We1, be1)
        h, Ps, Pd = _node(
            h, agg1, agg2,
            Wu1[:ind], Wu1[ind:], bu1, Wu2, bu2, g, Wg, bg,
            Wn1[:H], Wn1[H:], bn1)

    Fa1, Fb1 = _gather2(Ps, Pd, sidx1, didx1)
    Fa2, Fb2 = _gather2(Ps, Pd, sidx2, didx2)
    Wbig = jnp.kron(jnp.eye(B, dtype=f32), We2)
    rates_eb = jnp.concatenate(
        [_erate(Fa1, Fb1, Wbig, be2), _erate(Fa2, Fb2, Wbig, be2)], axis=0)
    rates_be = rates_eb.T.reshape(-1)

    key = src * N + dst
    loser = _dedup(key)
    rs = _rowsum(src.reshape(1, E), rates_eb, loser)
    dvals = (-rs).T.reshape(-1)

    bidx = jnp.arange(B, dtype=i32)
    redirect = (loser[:, 0] != 0) | (src == dst)
    flat = bidx[:, None] * (N * N) + key[None, :]
    fidx = jnp.where(redirect[None, :], TOT, flat).reshape(BE // 128, 128)
    diag_pos = jnp.arange(N, dtype=i32) * (N + 1)
    didx = (bidx[:, None] * (N * N) + diag_pos[None, :]).reshape(BN // 128, 128)

    rate_flat = _scatter(rates_be, fidx, dvals, didx, zeros1d)
    return rate_flat[:TOT].reshape(B, N, N)
